# baseline (speedup 1.0000x reference)
import functools

import jax
import jax.numpy as jnp
from jax import lax
from jax.experimental import pallas as pl
from jax.experimental.pallas import tpu as pltpu
from jax.experimental.pallas import tpu_sc as plsc

_NPAD = 10240
_CH = 128
_NW = 32
_LANES = 16
_NB = 8
_GL = 4
_SPLIT_AGG1 = (80, 80)
_SPLIT_AGG2 = (80, 80)
_ROWB = 2048
_SC_PARAMS = pltpu.CompilerParams(use_tc_tiling_on_sc=False)


def _fill(buf, ch, d, value):
    @pl.loop(0, ch)
    def _(r):
        @pl.loop(0, d // _LANES)
        def _(j):
            buf[r, pl.ds(j * _LANES, _LANES)] = jnp.full(
                (_LANES,), value, jnp.float32)


def _sc_degree(e3, npad, d, cpw0, cpw1):
    cpw_max = max(cpw0, cpw1)
    win = 16
    mesh = plsc.VectorSubcoreMesh(core_axis_name="c", subcore_axis_name="s")

    @functools.partial(
        pl.kernel,
        out_type=jax.ShapeDtypeStruct((2, npad, d), jnp.float32),
        mesh=mesh,
        scratch_types=[
            pltpu.VMEM((cpw_max, 2, _CH), jnp.int32),
            pltpu.VMEM((_CH, d), jnp.float32),
            pltpu.VMEM((_CH, d), jnp.float32),
            pltpu.VMEM_SHARED((npad, d), jnp.float32),
            pltpu.SemaphoreType.DMA,
            pltpu.SemaphoreType.DMA,
            pltpu.SemaphoreType.DMA,
            pltpu.SemaphoreType.DMA,
        ],
        compiler_params=_SC_PARAMS,
    )
    def k(e3_hbm, out_hbm, e3_v, zbuf, obuf, acc_sh, isem, zsem, ssem, osem):
        c = lax.axis_index("c")
        s = lax.axis_index("s")
        rpt = npad // 16
        mycpw = jnp.where(c == 0, cpw0, cpw1)
        base = jnp.where(c == 0, s * cpw0, 16 * cpw0 + s * cpw1)

        pltpu.async_copy(e3_hbm.at[pl.ds(base, cpw_max)], e3_v, isem)
        _fill(zbuf, _CH, d, 0.0)
        _fill(obuf, _CH, d, 1.0)
        for q in range(rpt // _CH):
            pltpu.async_copy(
                zbuf, acc_sh.at[pl.ds(s * rpt + q * _CH, _CH)], zsem)
        pltpu.make_async_copy(e3_hbm.at[pl.ds(base, cpw_max)], e3_v,
                              isem).wait()
        for q in range(rpt // _CH):
            pltpu.make_async_copy(
                zbuf, acc_sh.at[pl.ds(s * rpt + q * _CH, _CH)], zsem).wait()
        plsc.subcore_barrier()

        for t in range(win):
            pltpu.async_copy(obuf, acc_sh.at[e3_v.at[t, 1]], ssem, add=True)

        @pl.loop(win, mycpw)
        def _(t):
            pltpu.make_async_copy(obuf, acc_sh.at[e3_v.at[t, 1]], ssem).wait()
            pltpu.async_copy(obuf, acc_sh.at[e3_v.at[t, 1]], ssem, add=True)

        for t in range(win):
            pltpu.make_async_copy(obuf, acc_sh.at[e3_v.at[t, 1]], ssem).wait()
        plsc.subcore_barrier()

        for q in range(rpt // _CH):
            r0 = s * rpt + q * _CH
            pltpu.async_copy(acc_sh.at[pl.ds(r0, _CH)],
                             out_hbm.at[c, pl.ds(r0, _CH)], osem)
        for q in range(rpt // _CH):
            r0 = s * rpt + q * _CH
            pltpu.make_async_copy(acc_sh.at[pl.ds(r0, _CH)],
                                  out_hbm.at[c, pl.ds(r0, _CH)], osem).wait()

    return k(e3)


def _sc_edge_agg(table, e3, cpw0, cpw1, _NB, _GL):
    npad, d = table.shape
    cpw_max = max(cpw0, cpw1)
    assert cpw0 % _NB == 0 and cpw0 // _NB >= 2
    assert cpw1 % _NB == 0 and cpw1 // _NB >= 2
    assert min(cpw0, cpw1) >= _NB + _GL
    mesh = plsc.VectorSubcoreMesh(core_axis_name="c", subcore_axis_name="s")

    @functools.partial(
        pl.kernel,
        out_type=jax.ShapeDtypeStruct((2, npad, d), jnp.float32),
        mesh=mesh,
        scratch_types=(
            [pltpu.VMEM((cpw_max, 2, _CH), jnp.int32)]
            + [pltpu.VMEM((_CH, d), jnp.float32)] * _NB
            + [pltpu.VMEM_SHARED((npad, d), jnp.float32)]
            + [pltpu.SemaphoreType.DMA] * 4
            + [pltpu.SemaphoreType.DMA] * _NB
        ),
        compiler_params=_SC_PARAMS,
    )
    def k(table_hbm, e3_hbm, out_hbm, e3_v, *rest):
        rb = rest[:_NB]
        acc_sh = rest[_NB]
        isem, zsem, gsem, osem = rest[_NB + 1:_NB + 5]
        ssems = rest[_NB + 5:]
        c = lax.axis_index("c")
        s = lax.axis_index("s")
        rpt = npad // 16
        mycpw = jnp.where(c == 0, cpw0, cpw1)
        base = jnp.where(c == 0, s * cpw0, 16 * cpw0 + s * cpw1)

        pltpu.async_copy(e3_hbm.at[pl.ds(base, cpw_max)], e3_v, isem)
        _fill(rb[0], _CH, d, 0.0)
        for q in range(rpt // _CH):
            pltpu.async_copy(
                rb[0], acc_sh.at[pl.ds(s * rpt + q * _CH, _CH)], zsem)
        pltpu.make_async_copy(e3_hbm.at[pl.ds(base, cpw_max)], e3_v,
                              isem).wait()
        for q in range(rpt // _CH):
            pltpu.make_async_copy(
                rb[0], acc_sh.at[pl.ds(s * rpt + q * _CH, _CH)], zsem).wait()
        plsc.subcore_barrier()

        def issue_g(t, j):
            pltpu.async_copy(table_hbm.at[e3_v.at[t, 0]], rb[j], gsem)

        def wait_g(t, j):
            pltpu.make_async_copy(table_hbm.at[e3_v.at[t, 0]], rb[j],
                                  gsem).wait()

        def issue_s(t, j):
            pltpu.async_copy(rb[j], acc_sh.at[e3_v.at[t, 1]], ssems[j],
                             add=True)

        def wait_s(t, j):
            pltpu.make_async_copy(rb[j], acc_sh.at[e3_v.at[t, 1]],
                                  ssems[j]).wait()

        for j in range(_GL):
            issue_g(j, j)

        for j in range(_NB):
            wait_g(j, j)
            issue_s(j, j)
            u = j + _GL
            bu = u % _NB
            if u >= _NB:
                wait_s(u - _NB, bu)
            issue_g(u, bu)

        @pl.loop(1, mycpw // _NB - 1)
        def _(t8):
            for j in range(_NB):
                t = t8 * _NB + j
                wait_g(t, j)
                issue_s(t, j)
                bu = (j + _GL) % _NB
                wait_s(t + _GL - _NB, bu)
                issue_g(t + _GL, bu)

        tail = mycpw - _NB
        for j in range(_NB):
            t = tail + j
            wait_g(t, j)
            issue_s(t, j)
            if j < _NB - _GL:
                bu = (j + _GL) % _NB
                wait_s(t + _GL - _NB, bu)
                issue_g(t + _GL, bu)

        for j in range(_NB):
            wait_s(tail + j, j)
        plsc.subcore_barrier()

        for q in range(rpt // _CH):
            r0 = s * rpt + q * _CH
            pltpu.async_copy(acc_sh.at[pl.ds(r0, _CH)],
                             out_hbm.at[c, pl.ds(r0, _CH)], osem)
        for q in range(rpt // _CH):
            r0 = s * rpt + q * _CH
            pltpu.make_async_copy(acc_sh.at[pl.ds(r0, _CH)],
                                  out_hbm.at[c, pl.ds(r0, _CH)], osem).wait()

    return k(table, e3)


def _tc_mm_scale(deg_parts, x, w, npad):
    n, kdim = x.shape
    dh = w.shape[1]

    def body(dp_hbm, x_ref, w_ref, dinv_ref, hs_ref, dp_ref, sem):
        pltpu.async_copy(dp_hbm, dp_ref, sem).wait()
        deg = 1.0 + dp_ref[0, :, 0:1] + dp_ref[1, :, 0:1]
        dinv = lax.rsqrt(deg)
        dinv_ref[...] = jnp.broadcast_to(dinv, (npad, dh))
        h = jnp.dot(x_ref[...], w_ref[...],
                    preferred_element_type=jnp.float32)
        hs_ref[pl.ds(0, n), :] = h * dinv[:n]
        hs_ref[pl.ds(n, npad - n), :] = jnp.zeros((npad - n, dh), jnp.float32)

    return pl.pallas_call(
        body,
        in_specs=[
            pl.BlockSpec(memory_space=pl.ANY),
            pl.BlockSpec(x.shape, lambda: (0, 0)),
            pl.BlockSpec(w.shape, lambda: (0, 0)),
        ],
        scratch_shapes=[
            pltpu.VMEM(deg_parts.shape, jnp.float32),
            pltpu.SemaphoreType.DMA,
        ],
        out_shape=[
            jax.ShapeDtypeStruct((npad, dh), jnp.float32),
            jax.ShapeDtypeStruct((npad, dh), jnp.float32),
        ],
    )(deg_parts, x, w)


def _tc_layer2(agg1, hs1, dinv, b1, w2, n_real):
    n, dh = hs1.shape
    dout = w2.shape[1]

    def body(ag_hbm, h_ref, dv_ref, b_ref, w_ref, hs2_ref, ag_ref, sem):
        pltpu.async_copy(ag_hbm, ag_ref, sem).wait()
        a = ag_ref[0] + ag_ref[1]
        dinv_b = dv_ref[...]
        pre = dinv_b * (a + h_ref[...]) + b_ref[...]
        out1 = jnp.maximum(pre, 0.0)
        row = lax.broadcasted_iota(jnp.int32, (n, 1), 0)
        out1 = jnp.where(row < n_real, out1, 0.0)
        h2 = jnp.dot(out1, w_ref[...], preferred_element_type=jnp.float32)
        hs2_ref[...] = h2 * dv_ref[:, :dout]

    return pl.pallas_call(
        body,
        in_specs=[
            pl.BlockSpec(memory_space=pl.ANY),
            pl.BlockSpec(hs1.shape, lambda: (0, 0)),
            pl.BlockSpec(dinv.shape, lambda: (0, 0)),
            pl.BlockSpec(b1.shape, lambda: (0, 0)),
            pl.BlockSpec(w2.shape, lambda: (0, 0)),
        ],
        scratch_shapes=[
            pltpu.VMEM(agg1.shape, jnp.float32),
            pltpu.SemaphoreType.DMA,
        ],
        out_shape=jax.ShapeDtypeStruct((n, dout), jnp.float32),
    )(agg1, hs1, dinv, b1, w2)


def _tc_final(agg2, hs2, dinv, b2):
    n, dout = hs2.shape

    def body(ag_hbm, h_ref, dv_ref, b_ref, o_ref, ag_ref, sem):
        pltpu.async_copy(ag_hbm, ag_ref, sem).wait()
        a = ag_ref[0] + ag_ref[1]
        dinv_b = dv_ref[:, :dout]
        o = dinv_b * (a + h_ref[...]) + b_ref[...]
        m = jnp.max(o, axis=1, keepdims=True)
        e = jnp.exp(o - m)
        lse = jnp.log(jnp.sum(e, axis=1, keepdims=True)) + m
        o_ref[...] = o - lse

    return pl.pallas_call(
        body,
        in_specs=[
            pl.BlockSpec(memory_space=pl.ANY),
            pl.BlockSpec(hs2.shape, lambda: (0, 0)),
            pl.BlockSpec(dinv.shape, lambda: (0, 0)),
            pl.BlockSpec(b2.shape, lambda: (0, 0)),
        ],
        scratch_shapes=[
            pltpu.VMEM(agg2.shape, jnp.float32),
            pltpu.SemaphoreType.DMA,
        ],
        out_shape=jax.ShapeDtypeStruct((n, dout), jnp.float32),
    )(agg2, hs2, dinv, b2)


def kernel(x, edge_index, W1, b1, W2, b2):
    n, d_in = x.shape
    e = edge_index.shape[1]

    assert sum(_SPLIT_AGG1) == sum(_SPLIT_AGG2)
    nchunks = 16 * sum(_SPLIT_AGG1)
    assert nchunks * _CH >= e
    arr_rows = nchunks + max(max(_SPLIT_AGG1), max(_SPLIT_AGG2))
    assert e % _CH == 0
    erows = e // _CH
    ei3 = jnp.transpose(edge_index.reshape(2, erows, _CH), (1, 0, 2))
    pad_idx = (n + jnp.arange((arr_rows - erows) * _CH, dtype=jnp.int32)
               % (_NPAD - n)).reshape(-1, _CH)
    e3 = jnp.concatenate(
        [ei3, jnp.broadcast_to(pad_idx[:, None, :],
                               (arr_rows - erows, 2, _CH))])
    deg_parts = _sc_degree(e3, _NPAD, 16, *_SPLIT_AGG2)
    dinv, hs1 = _tc_mm_scale(deg_parts, x, W1, _NPAD)
    agg1 = _sc_edge_agg(hs1, e3, *_SPLIT_AGG1, _NB, _GL)
    hs2 = _tc_layer2(agg1, hs1, dinv, b1.reshape(1, -1), W2, n)
    agg2 = _sc_edge_agg(hs2, e3, *_SPLIT_AGG2, 20, 10)
    out = _tc_final(agg2, hs2, dinv, b2.reshape(1, -1))
    return out[:n]


# speedup vs baseline: 1.0157x; 1.0157x over previous
import functools

import jax
import jax.numpy as jnp
from jax import lax
from jax.experimental import pallas as pl
from jax.experimental.pallas import tpu as pltpu
from jax.experimental.pallas import tpu_sc as plsc

_NPAD = 10240
_CH = 128
_NW = 32
_LANES = 16
_NB = 8
_GL = 4
_SPLIT_AGG1 = (80, 80)
_SPLIT_AGG2 = (80, 80)
_ROWB = 2048
_SC_PARAMS = pltpu.CompilerParams(use_tc_tiling_on_sc=False)


def _fill(buf, ch, d, value):
    @pl.loop(0, ch)
    def _(r):
        @pl.loop(0, d // _LANES)
        def _(j):
            buf[r, pl.ds(j * _LANES, _LANES)] = jnp.full(
                (_LANES,), value, jnp.float32)


def _sc_degree(e3, npad, d, cpw0, cpw1):
    cpw_max = max(cpw0, cpw1)
    win = 16
    mesh = plsc.VectorSubcoreMesh(core_axis_name="c", subcore_axis_name="s")

    @functools.partial(
        pl.kernel,
        out_type=jax.ShapeDtypeStruct((2, npad, d), jnp.float32),
        mesh=mesh,
        scratch_types=[
            pltpu.VMEM((cpw_max, 2, _CH), jnp.int32),
            pltpu.VMEM((_CH, d), jnp.float32),
            pltpu.VMEM((_CH, d), jnp.float32),
            pltpu.VMEM_SHARED((npad, d), jnp.float32),
            pltpu.SemaphoreType.DMA,
            pltpu.SemaphoreType.DMA,
            pltpu.SemaphoreType.DMA,
            pltpu.SemaphoreType.DMA,
        ],
        compiler_params=_SC_PARAMS,
    )
    def k(e3_hbm, out_hbm, e3_v, zbuf, obuf, acc_sh, isem, zsem, ssem, osem):
        c = lax.axis_index("c")
        s = lax.axis_index("s")
        rpt = npad // 16
        mycpw = jnp.where(c == 0, cpw0, cpw1)
        base = jnp.where(c == 0, s * cpw0, 16 * cpw0 + s * cpw1)

        pltpu.async_copy(e3_hbm.at[pl.ds(base, cpw_max)], e3_v, isem)
        _fill(zbuf, _CH, d, 0.0)
        _fill(obuf, _CH, d, 1.0)
        for q in range(rpt // _CH):
            pltpu.async_copy(
                zbuf, acc_sh.at[pl.ds(s * rpt + q * _CH, _CH)], zsem)
        pltpu.make_async_copy(e3_hbm.at[pl.ds(base, cpw_max)], e3_v,
                              isem).wait()
        for q in range(rpt // _CH):
            pltpu.make_async_copy(
                zbuf, acc_sh.at[pl.ds(s * rpt + q * _CH, _CH)], zsem).wait()
        plsc.subcore_barrier()

        for t in range(win):
            pltpu.async_copy(obuf, acc_sh.at[e3_v.at[t, 1]], ssem, add=True)

        @pl.loop(win, mycpw)
        def _(t):
            pltpu.make_async_copy(obuf, acc_sh.at[e3_v.at[t, 1]], ssem).wait()
            pltpu.async_copy(obuf, acc_sh.at[e3_v.at[t, 1]], ssem, add=True)

        for t in range(win):
            pltpu.make_async_copy(obuf, acc_sh.at[e3_v.at[t, 1]], ssem).wait()
        plsc.subcore_barrier()

        for q in range(rpt // _CH):
            r0 = s * rpt + q * _CH
            pltpu.async_copy(acc_sh.at[pl.ds(r0, _CH)],
                             out_hbm.at[c, pl.ds(r0, _CH)], osem)
        for q in range(rpt // _CH):
            r0 = s * rpt + q * _CH
            pltpu.make_async_copy(acc_sh.at[pl.ds(r0, _CH)],
                                  out_hbm.at[c, pl.ds(r0, _CH)], osem).wait()

    return k(e3)


def _sc_edge_agg(table, e3, cpw0, cpw1, _NB, _GL):
    npad, d = table.shape
    cpw_max = max(cpw0, cpw1)
    assert cpw0 % _NB == 0 and cpw0 // _NB >= 2
    assert cpw1 % _NB == 0 and cpw1 // _NB >= 2
    assert min(cpw0, cpw1) >= _NB + _GL
    mesh = plsc.VectorSubcoreMesh(core_axis_name="c", subcore_axis_name="s")

    @functools.partial(
        pl.kernel,
        out_type=jax.ShapeDtypeStruct((2, npad, d), jnp.float32),
        mesh=mesh,
        scratch_types=(
            [pltpu.VMEM((cpw_max, 2, _CH), jnp.int32)]
            + [pltpu.VMEM((_CH, d), jnp.float32)] * _NB
            + [pltpu.VMEM_SHARED((npad, d), jnp.float32)]
            + [pltpu.SemaphoreType.DMA] * 4
            + [pltpu.SemaphoreType.DMA] * _NB
        ),
        compiler_params=_SC_PARAMS,
    )
    def k(table_hbm, e3_hbm, out_hbm, e3_v, *rest):
        rb = rest[:_NB]
        acc_sh = rest[_NB]
        isem, zsem, gsem, osem = rest[_NB + 1:_NB + 5]
        ssems = rest[_NB + 5:]
        c = lax.axis_index("c")
        s = lax.axis_index("s")
        rpt = npad // 16
        mycpw = jnp.where(c == 0, cpw0, cpw1)
        base = jnp.where(c == 0, s * cpw0, 16 * cpw0 + s * cpw1)

        pltpu.async_copy(e3_hbm.at[pl.ds(base, cpw_max)], e3_v, isem)
        _fill(rb[0], _CH, d, 0.0)
        for q in range(rpt // _CH):
            pltpu.async_copy(
                rb[0], acc_sh.at[pl.ds(s * rpt + q * _CH, _CH)], zsem)
        pltpu.make_async_copy(e3_hbm.at[pl.ds(base, cpw_max)], e3_v,
                              isem).wait()
        for q in range(rpt // _CH):
            pltpu.make_async_copy(
                rb[0], acc_sh.at[pl.ds(s * rpt + q * _CH, _CH)], zsem).wait()
        plsc.subcore_barrier()

        def issue_g(t, j):
            pltpu.async_copy(table_hbm.at[e3_v.at[t, 0]], rb[j], gsem)

        def wait_g(t, j):
            pltpu.make_async_copy(table_hbm.at[e3_v.at[t, 0]], rb[j],
                                  gsem).wait()

        def issue_s(t, j):
            pltpu.async_copy(rb[j], acc_sh.at[e3_v.at[t, 1]], ssems[j],
                             add=True)

        def wait_s(t, j):
            pltpu.make_async_copy(rb[j], acc_sh.at[e3_v.at[t, 1]],
                                  ssems[j]).wait()

        for j in range(_GL):
            issue_g(j, j)

        for j in range(_NB):
            wait_g(j, j)
            issue_s(j, j)
            u = j + _GL
            bu = u % _NB
            if u >= _NB:
                wait_s(u - _NB, bu)
            issue_g(u, bu)

        @pl.loop(1, mycpw // _NB - 1)
        def _(t8):
            for j in range(_NB):
                t = t8 * _NB + j
                wait_g(t, j)
                issue_s(t, j)
                bu = (j + _GL) % _NB
                wait_s(t + _GL - _NB, bu)
                issue_g(t + _GL, bu)

        tail = mycpw - _NB
        for j in range(_NB):
            t = tail + j
            wait_g(t, j)
            issue_s(t, j)
            if j < _NB - _GL:
                bu = (j + _GL) % _NB
                wait_s(t + _GL - _NB, bu)
                issue_g(t + _GL, bu)

        for j in range(_NB):
            wait_s(tail + j, j)
        plsc.subcore_barrier()

        for q in range(rpt // _CH):
            r0 = s * rpt + q * _CH
            pltpu.async_copy(acc_sh.at[pl.ds(r0, _CH)],
                             out_hbm.at[c, pl.ds(r0, _CH)], osem)
        for q in range(rpt // _CH):
            r0 = s * rpt + q * _CH
            pltpu.make_async_copy(acc_sh.at[pl.ds(r0, _CH)],
                                  out_hbm.at[c, pl.ds(r0, _CH)], osem).wait()

    return k(table, e3)


def _tc_mm_scale(deg_parts, x, w, npad):
    n, kdim = x.shape
    dh = w.shape[1]

    def body(dp_ref, x_ref, w_ref, dinv_ref, hs_ref):
        deg = 1.0 + dp_ref[0, :, 0:1] + dp_ref[1, :, 0:1]
        dinv = lax.rsqrt(deg)
        dinv_ref[...] = jnp.broadcast_to(dinv, (npad, dh))
        h = jnp.dot(x_ref[...], w_ref[...],
                    preferred_element_type=jnp.float32)
        hs_ref[pl.ds(0, n), :] = h * dinv[:n]
        hs_ref[pl.ds(n, npad - n), :] = jnp.zeros((npad - n, dh), jnp.float32)

    return pl.pallas_call(
        body,
        out_shape=[
            jax.ShapeDtypeStruct((npad, dh), jnp.float32),
            jax.ShapeDtypeStruct((npad, dh), jnp.float32),
        ],
    )(deg_parts, x, w)


def _tc_layer2(agg1, hs1, dinv, b1, w2, n_real):
    n, dh = hs1.shape
    dout = w2.shape[1]

    def body(ag_ref, h_ref, dv_ref, b_ref, w_ref, hs2_ref):
        a = ag_ref[0] + ag_ref[1]
        dinv_b = dv_ref[...]
        pre = dinv_b * (a + h_ref[...]) + b_ref[...]
        out1 = jnp.maximum(pre, 0.0)
        row = lax.broadcasted_iota(jnp.int32, (n, 1), 0)
        out1 = jnp.where(row < n_real, out1, 0.0)
        h2 = jnp.dot(out1, w_ref[...], preferred_element_type=jnp.float32)
        hs2_ref[...] = h2 * dv_ref[:, :dout]

    return pl.pallas_call(
        body,
        out_shape=jax.ShapeDtypeStruct((n, dout), jnp.float32),
    )(agg1, hs1, dinv, b1, w2)


def _tc_final(agg2, hs2, dinv, b2):
    n, dout = hs2.shape

    def body(ag_ref, h_ref, dv_ref, b_ref, o_ref):
        a = ag_ref[0] + ag_ref[1]
        dinv_b = dv_ref[:, :dout]
        o = dinv_b * (a + h_ref[...]) + b_ref[...]
        m = jnp.max(o, axis=1, keepdims=True)
        e = jnp.exp(o - m)
        lse = jnp.log(jnp.sum(e, axis=1, keepdims=True)) + m
        o_ref[...] = o - lse

    return pl.pallas_call(
        body,
        out_shape=jax.ShapeDtypeStruct((n, dout), jnp.float32),
    )(agg2, hs2, dinv, b2)


def kernel(x, edge_index, W1, b1, W2, b2):
    n, d_in = x.shape
    e = edge_index.shape[1]

    assert sum(_SPLIT_AGG1) == sum(_SPLIT_AGG2)
    nchunks = 16 * sum(_SPLIT_AGG1)
    assert nchunks * _CH >= e
    arr_rows = nchunks + max(max(_SPLIT_AGG1), max(_SPLIT_AGG2))
    assert e % _CH == 0
    erows = e // _CH
    ei3 = jnp.transpose(edge_index.reshape(2, erows, _CH), (1, 0, 2))
    pad_idx = (n + jnp.arange((arr_rows - erows) * _CH, dtype=jnp.int32)
               % (_NPAD - n)).reshape(-1, _CH)
    e3 = jnp.concatenate(
        [ei3, jnp.broadcast_to(pad_idx[:, None, :],
                               (arr_rows - erows, 2, _CH))])
    deg_parts = _sc_degree(e3, _NPAD, 16, *_SPLIT_AGG2)
    dinv, hs1 = _tc_mm_scale(deg_parts, x, W1, _NPAD)
    agg1 = _sc_edge_agg(hs1, e3, *_SPLIT_AGG1, _NB, _GL)
    hs2 = _tc_layer2(agg1, hs1, dinv, b1.reshape(1, -1), W2, n)
    agg2 = _sc_edge_agg(hs2, e3, *_SPLIT_AGG2, 20, 10)
    out = _tc_final(agg2, hs2, dinv, b2.reshape(1, -1))
    return out[:n]


# speedup vs baseline: 1.0161x; 1.0005x over previous
import functools

import jax
import jax.numpy as jnp
from jax import lax
from jax.experimental import pallas as pl
from jax.experimental.pallas import tpu as pltpu
from jax.experimental.pallas import tpu_sc as plsc

_NPAD = 10240
_CH = 128
_NW = 32
_LANES = 16
_NB = 8
_GL = 4
_SPLIT_AGG1 = (80, 80)
_SPLIT_AGG2 = (80, 80)
_SC_PARAMS = pltpu.CompilerParams(use_tc_tiling_on_sc=False)


def _fill(buf, ch, d, value):
    @pl.loop(0, ch)
    def _(r):
        @pl.loop(0, d // _LANES)
        def _(j):
            buf[r, pl.ds(j * _LANES, _LANES)] = jnp.full(
                (_LANES,), value, jnp.float32)


def _sc_degree(e3, npad, d, cpw0, cpw1):
    cpw_max = max(cpw0, cpw1)
    win = 16
    mesh = plsc.VectorSubcoreMesh(core_axis_name="c", subcore_axis_name="s")

    @functools.partial(
        pl.kernel,
        out_type=jax.ShapeDtypeStruct((2, npad, d), jnp.float32),
        mesh=mesh,
        scratch_types=[
            pltpu.VMEM((cpw_max, 2, _CH), jnp.int32),
            pltpu.VMEM((_CH, d), jnp.float32),
            pltpu.VMEM((_CH, d), jnp.float32),
            pltpu.VMEM_SHARED((npad, d), jnp.float32),
            pltpu.SemaphoreType.DMA,
            pltpu.SemaphoreType.DMA,
            pltpu.SemaphoreType.DMA,
            pltpu.SemaphoreType.DMA,
        ],
        compiler_params=_SC_PARAMS,
    )
    def k(e3_hbm, out_hbm, e3_v, zbuf, obuf, acc_sh, isem, zsem, ssem, osem):
        c = lax.axis_index("c")
        s = lax.axis_index("s")
        rpt = npad // 16
        mycpw = jnp.where(c == 0, cpw0, cpw1)
        base = jnp.where(c == 0, s * cpw0, 16 * cpw0 + s * cpw1)

        pltpu.async_copy(e3_hbm.at[pl.ds(base, cpw_max)], e3_v, isem)
        _fill(zbuf, _CH, d, 0.0)
        _fill(obuf, _CH, d, 1.0)
        for q in range(rpt // _CH):
            pltpu.async_copy(
                zbuf, acc_sh.at[pl.ds(s * rpt + q * _CH, _CH)], zsem)
        pltpu.make_async_copy(e3_hbm.at[pl.ds(base, cpw_max)], e3_v,
                              isem).wait()
        for q in range(rpt // _CH):
            pltpu.make_async_copy(
                zbuf, acc_sh.at[pl.ds(s * rpt + q * _CH, _CH)], zsem).wait()
        plsc.subcore_barrier()

        for t in range(win):
            pltpu.async_copy(obuf, acc_sh.at[e3_v.at[t, 1]], ssem, add=True)

        @pl.loop(win, mycpw)
        def _(t):
            pltpu.make_async_copy(obuf, acc_sh.at[e3_v.at[t, 1]], ssem).wait()
            pltpu.async_copy(obuf, acc_sh.at[e3_v.at[t, 1]], ssem, add=True)

        for t in range(win):
            pltpu.make_async_copy(obuf, acc_sh.at[e3_v.at[t, 1]], ssem).wait()
        plsc.subcore_barrier()

        for q in range(rpt // _CH):
            r0 = s * rpt + q * _CH
            pltpu.async_copy(acc_sh.at[pl.ds(r0, _CH)],
                             out_hbm.at[c, pl.ds(r0, _CH)], osem)
        for q in range(rpt // _CH):
            r0 = s * rpt + q * _CH
            pltpu.make_async_copy(acc_sh.at[pl.ds(r0, _CH)],
                                  out_hbm.at[c, pl.ds(r0, _CH)], osem).wait()

    return k(e3)


def _sc_edge_agg(table, e3, cpw0, cpw1, _NB, _GL):
    npad, d = table.shape
    cpw_max = max(cpw0, cpw1)
    assert cpw0 % _NB == 0 and cpw0 // _NB >= 2
    assert cpw1 % _NB == 0 and cpw1 // _NB >= 2
    assert min(cpw0, cpw1) >= _NB + _GL
    mesh = plsc.VectorSubcoreMesh(core_axis_name="c", subcore_axis_name="s")

    @functools.partial(
        pl.kernel,
        out_type=jax.ShapeDtypeStruct((2, npad, d), jnp.float32),
        mesh=mesh,
        scratch_types=(
            [pltpu.VMEM((cpw_max, 2, _CH), jnp.int32)]
            + [pltpu.VMEM((_CH, d), jnp.float32)] * _NB
            + [pltpu.VMEM_SHARED((npad, d), jnp.float32)]
            + [pltpu.SemaphoreType.DMA] * 4
            + [pltpu.SemaphoreType.DMA] * _NB
        ),
        compiler_params=_SC_PARAMS,
    )
    def k(table_hbm, e3_hbm, out_hbm, e3_v, *rest):
        rb = rest[:_NB]
        acc_sh = rest[_NB]
        isem, zsem, gsem, osem = rest[_NB + 1:_NB + 5]
        ssems = rest[_NB + 5:]
        c = lax.axis_index("c")
        s = lax.axis_index("s")
        rpt = npad // 16
        mycpw = jnp.where(c == 0, cpw0, cpw1)
        base = jnp.where(c == 0, s * cpw0, 16 * cpw0 + s * cpw1)

        pltpu.async_copy(e3_hbm.at[pl.ds(base, cpw_max)], e3_v, isem)
        _fill(rb[0], _CH, d, 0.0)
        for q in range(rpt // _CH):
            pltpu.async_copy(
                rb[0], acc_sh.at[pl.ds(s * rpt + q * _CH, _CH)], zsem)
        pltpu.make_async_copy(e3_hbm.at[pl.ds(base, cpw_max)], e3_v,
                              isem).wait()
        for q in range(rpt // _CH):
            pltpu.make_async_copy(
                rb[0], acc_sh.at[pl.ds(s * rpt + q * _CH, _CH)], zsem).wait()
        plsc.subcore_barrier()

        def issue_g(t, j):
            pltpu.async_copy(table_hbm.at[e3_v.at[t, 0]], rb[j], gsem)

        def wait_g(t, j):
            pltpu.make_async_copy(table_hbm.at[e3_v.at[t, 0]], rb[j],
                                  gsem).wait()

        def issue_s(t, j):
            pltpu.async_copy(rb[j], acc_sh.at[e3_v.at[t, 1]], ssems[j],
                             add=True)

        def wait_s(t, j):
            pltpu.make_async_copy(rb[j], acc_sh.at[e3_v.at[t, 1]],
                                  ssems[j]).wait()

        for j in range(_GL):
            issue_g(j, j)

        for j in range(_NB):
            wait_g(j, j)
            issue_s(j, j)
            u = j + _GL
            bu = u % _NB
            if u >= _NB:
                wait_s(u - _NB, bu)
            issue_g(u, bu)

        @pl.loop(1, mycpw // _NB - 1)
        def _(t8):
            for j in range(_NB):
                t = t8 * _NB + j
                wait_g(t, j)
                issue_s(t, j)
                bu = (j + _GL) % _NB
                wait_s(t + _GL - _NB, bu)
                issue_g(t + _GL, bu)

        tail = mycpw - _NB
        for j in range(_NB):
            t = tail + j
            wait_g(t, j)
            issue_s(t, j)
            if j < _NB - _GL:
                bu = (j + _GL) % _NB
                wait_s(t + _GL - _NB, bu)
                issue_g(t + _GL, bu)

        for j in range(_NB):
            wait_s(tail + j, j)
        plsc.subcore_barrier()

        for q in range(rpt // _CH):
            r0 = s * rpt + q * _CH
            pltpu.async_copy(acc_sh.at[pl.ds(r0, _CH)],
                             out_hbm.at[c, pl.ds(r0, _CH)], osem)
        for q in range(rpt // _CH):
            r0 = s * rpt + q * _CH
            pltpu.make_async_copy(acc_sh.at[pl.ds(r0, _CH)],
                                  out_hbm.at[c, pl.ds(r0, _CH)], osem).wait()

    return k(table, e3)


def _tc_mm_scale(deg_parts, x, w, npad):
    n, kdim = x.shape
    dh = w.shape[1]

    def body(dp_ref, x_ref, w_ref, dinv_ref, hs_ref):
        deg = 1.0 + dp_ref[0, :, 0:1] + dp_ref[1, :, 0:1]
        dinv = lax.rsqrt(deg)
        dinv_ref[...] = jnp.broadcast_to(dinv, (npad, dh))
        h = jnp.dot(x_ref[...], w_ref[...],
                    preferred_element_type=jnp.float32)
        hs_ref[pl.ds(0, n), :] = h * dinv[:n]
        hs_ref[pl.ds(n, npad - n), :] = jnp.zeros((npad - n, dh), jnp.float32)

    return pl.pallas_call(
        body,
        out_shape=[
            jax.ShapeDtypeStruct((npad, dh), jnp.float32),
            jax.ShapeDtypeStruct((npad, dh), jnp.float32),
        ],
    )(deg_parts, x, w)


def _tc_layer2(agg1, hs1, dinv, b1, w2, n_real):
    n, dh = hs1.shape
    dout = w2.shape[1]

    def body(ag_ref, h_ref, dv_ref, b_ref, w_ref, hs2_ref):
        a = ag_ref[0] + ag_ref[1]
        dinv_b = dv_ref[...]
        pre = dinv_b * (a + h_ref[...]) + b_ref[...]
        out1 = jnp.maximum(pre, 0.0)
        row = lax.broadcasted_iota(jnp.int32, (n, 1), 0)
        out1 = jnp.where(row < n_real, out1, 0.0)
        h2 = jnp.dot(out1, w_ref[...], preferred_element_type=jnp.float32)
        hs2_ref[...] = h2 * dv_ref[:, :dout]

    return pl.pallas_call(
        body,
        out_shape=jax.ShapeDtypeStruct((n, dout), jnp.float32),
    )(agg1, hs1, dinv, b1, w2)


def _tc_final(agg2, hs2, dinv, b2):
    n, dout = hs2.shape

    def body(ag_ref, h_ref, dv_ref, b_ref, o_ref):
        a = ag_ref[0] + ag_ref[1]
        dinv_b = dv_ref[:, :dout]
        o = dinv_b * (a + h_ref[...]) + b_ref[...]
        m = jnp.max(o, axis=1, keepdims=True)
        e = jnp.exp(o - m)
        lse = jnp.log(jnp.sum(e, axis=1, keepdims=True)) + m
        o_ref[...] = o - lse

    return pl.pallas_call(
        body,
        out_shape=jax.ShapeDtypeStruct((n, dout), jnp.float32),
    )(agg2, hs2, dinv, b2)


def kernel(x, edge_index, W1, b1, W2, b2):
    n, d_in = x.shape
    e = edge_index.shape[1]

    assert sum(_SPLIT_AGG1) == sum(_SPLIT_AGG2)
    nchunks = 16 * sum(_SPLIT_AGG1)
    assert nchunks * _CH >= e
    arr_rows = nchunks + max(max(_SPLIT_AGG1), max(_SPLIT_AGG2))
    assert e % _CH == 0
    erows = e // _CH
    ei3 = jnp.transpose(edge_index.reshape(2, erows, _CH), (1, 0, 2))
    pad_idx = (n + jnp.arange((arr_rows - erows) * _CH, dtype=jnp.int32)
               % (_NPAD - n)).reshape(-1, _CH)
    e3 = jnp.concatenate(
        [ei3, jnp.broadcast_to(pad_idx[:, None, :],
                               (arr_rows - erows, 2, _CH))])
    deg_parts = _sc_degree(e3, _NPAD, 16, *_SPLIT_AGG2)
    dinv, hs1 = _tc_mm_scale(deg_parts, x, W1, _NPAD)
    agg1 = _sc_edge_agg(hs1, e3, *_SPLIT_AGG1, _NB, _GL)
    hs2 = _tc_layer2(agg1, hs1, dinv, b1.reshape(1, -1), W2, n)
    agg2 = _sc_edge_agg(hs2, e3, *_SPLIT_AGG2, 20, 10)
    out = _tc_final(agg2, hs2, dinv, b2.reshape(1, -1))
    return out[:n]
